# Initial kernel scaffold; baseline (speedup 1.0000x reference)
import jax
import jax.numpy as jnp
from jax.experimental import pallas as pl


def kernel(x, edge_index, edge_attr, u, batch):
    raise NotImplementedError("write your pallas kernel here")


# speedup vs baseline: 5.3502x; 5.3502x over previous
import functools

import jax
import jax.numpy as jnp
from jax import lax
from jax.experimental import pallas as pl
from jax.experimental.pallas import tpu as pltpu
from jax.experimental.pallas import tpu_sc as plsc

N_NODES = 10000
N_EDGES = 320000
D_EDGE = 16
N_PAD = 10240

NC = 2
NS = 16
NW = NC * NS

B = 80
G_TOTAL = N_EDGES // B
CHUNK_G = 32
CHUNK_E = CHUNK_G * B
N_CHUNKS = G_TOTAL // CHUNK_G
MAX_ROUNDS = -(-N_CHUNKS // NW)
RPS = N_PAD // NS

_mesh = plsc.VectorSubcoreMesh(core_axis_name="c", subcore_axis_name="s")


@functools.partial(
    pl.kernel,
    out_type=jax.ShapeDtypeStruct((NC, N_PAD, D_EDGE), jnp.float32),
    mesh=_mesh,
    compiler_params=pltpu.CompilerParams(use_tc_tiling_on_sc=False),
    scratch_types=[
        pltpu.VMEM((CHUNK_G, B), jnp.int32),
        pltpu.VMEM((CHUNK_E, D_EDGE), jnp.float32),
        pltpu.VMEM((RPS, D_EDGE), jnp.float32),
        pltpu.VMEM_SHARED((N_PAD, D_EDGE), jnp.float32),
    ],
)
def _scatter_partials(col2d, attr, partials, idx_v, attr_v, zbuf, acc):
    c = lax.axis_index("c")
    s = lax.axis_index("s")
    w = s * NC + c

    zero16 = jnp.zeros((D_EDGE,), jnp.float32)

    def _zero(i, carry):
        zbuf[i, :] = zero16
        return carry

    lax.fori_loop(0, RPS, _zero, 0)
    pltpu.sync_copy(zbuf, acc.at[pl.ds(s * RPS, RPS), :])
    plsc.subcore_barrier()

    for rnd in range(MAX_ROUNDS):
        t = rnd * NW + w

        @pl.when(t < N_CHUNKS)
        def _():
            g0 = t * CHUNK_G
            e0 = t * CHUNK_E
            pltpu.sync_copy(col2d.at[pl.ds(g0, CHUNK_G), :], idx_v)
            pltpu.sync_copy(attr.at[pl.ds(e0, CHUNK_E), :], attr_v)

            def _scat(j, carry):
                pltpu.sync_copy(attr_v.at[pl.ds(j * B, B), :],
                                acc.at[idx_v.at[j]], add=True)
                return carry

            lax.fori_loop(0, CHUNK_G, _scat, 0)

    plsc.subcore_barrier()
    pltpu.sync_copy(acc.at[pl.ds(s * RPS, RPS), :],
                    partials.at[c, pl.ds(s * RPS, RPS), :])


CR = N_PAD // NW


@functools.partial(
    pl.kernel,
    out_type=jax.ShapeDtypeStruct((N_PAD, D_EDGE), jnp.float32),
    mesh=_mesh,
    compiler_params=pltpu.CompilerParams(use_tc_tiling_on_sc=False),
    scratch_types=[
        pltpu.VMEM((CR, D_EDGE), jnp.float32),
        pltpu.VMEM((CR, D_EDGE), jnp.float32),
    ],
)
def _combine(partials, out, a_v, b_v):
    c = lax.axis_index("c")
    s = lax.axis_index("s")
    w = s * NC + c
    r0 = w * CR

    pltpu.sync_copy(partials.at[0, pl.ds(r0, CR), :], a_v)
    pltpu.sync_copy(partials.at[1, pl.ds(r0, CR), :], b_v)

    def _add(i, carry):
        a_v[i, :] = a_v[i, :] + b_v[i, :]
        return carry

    lax.fori_loop(0, CR, _add, 0)
    pltpu.sync_copy(a_v, out.at[pl.ds(r0, CR), :])


def kernel(x, edge_index, edge_attr, u, batch):
    col = edge_index[1].astype(jnp.int32).reshape(G_TOTAL, B)
    partials = _scatter_partials(col, edge_attr)
    return _combine(partials)[:N_NODES]


# speedup vs baseline: 5.7909x; 1.0824x over previous
import functools

import jax
import jax.numpy as jnp
from jax import lax
from jax.experimental import pallas as pl
from jax.experimental.pallas import tpu as pltpu
from jax.experimental.pallas import tpu_sc as plsc

N_NODES = 10000
N_EDGES = 320000
D_EDGE = 16
N_PAD = 10240

NC = 2
NS = 16
NW = NC * NS

B = 80
G_TOTAL = N_EDGES // B
CHUNK_G = 32
CHUNK_E = CHUNK_G * B
N_CHUNKS = G_TOTAL // CHUNK_G
MAX_ROUNDS = -(-N_CHUNKS // NW)
RPS = N_PAD // NS

_mesh = plsc.VectorSubcoreMesh(core_axis_name="c", subcore_axis_name="s")


@functools.partial(
    pl.kernel,
    out_type=jax.ShapeDtypeStruct((NC, N_PAD, D_EDGE), jnp.float32),
    mesh=_mesh,
    compiler_params=pltpu.CompilerParams(use_tc_tiling_on_sc=False),
    scratch_types=[
        pltpu.VMEM((2, CHUNK_G, B), jnp.int32),
        pltpu.VMEM((2, CHUNK_E, D_EDGE), jnp.float32),
        pltpu.VMEM((RPS, D_EDGE), jnp.float32),
        pltpu.VMEM_SHARED((N_PAD, D_EDGE), jnp.float32),
        pltpu.SemaphoreType.DMA,
        pltpu.SemaphoreType.DMA,
        pltpu.SemaphoreType.DMA,
    ],
)
def _scatter_partials(col2d, attr, partials, idx_v, attr_v, zbuf, acc,
                      sem_l0, sem_l1, sem_s):
    c = lax.axis_index("c")
    s = lax.axis_index("s")
    w = s * NC + c
    sem_l = (sem_l0, sem_l1)

    zero16 = jnp.zeros((D_EDGE,), jnp.float32)

    def _zero(i, carry):
        zbuf[i, :] = zero16
        return carry

    lax.fori_loop(0, RPS, _zero, 0)
    pltpu.sync_copy(zbuf, acc.at[pl.ds(s * RPS, RPS), :])
    plsc.subcore_barrier()

    def _fire_loads(rnd, buf):
        t = rnd * NW + w

        @pl.when(t < N_CHUNKS)
        def _():
            pltpu.async_copy(col2d.at[pl.ds(t * CHUNK_G, CHUNK_G), :],
                             idx_v.at[buf], sem_l[buf])
            pltpu.async_copy(attr.at[pl.ds(t * CHUNK_E, CHUNK_E), :],
                             attr_v.at[buf], sem_l[buf])

    _fire_loads(0, 0)
    for rnd in range(MAX_ROUNDS):
        b = rnd % 2
        t = rnd * NW + w
        if rnd + 1 < MAX_ROUNDS:
            _fire_loads(rnd + 1, 1 - b)

        @pl.when(t < N_CHUNKS)
        def _():
            pltpu.make_async_copy(col2d.at[pl.ds(0, CHUNK_G), :],
                                  idx_v.at[b], sem_l[b]).wait()
            pltpu.make_async_copy(attr.at[pl.ds(0, CHUNK_E), :],
                                  attr_v.at[b], sem_l[b]).wait()

            def _scat(j, carry):
                pltpu.async_copy(attr_v.at[b, pl.ds(j * B, B), :],
                                 acc.at[idx_v.at[b, j]], sem_s, add=True)
                return carry

            lax.fori_loop(0, CHUNK_G, _scat, 0)
            pltpu.make_async_copy(attr.at[pl.ds(0, CHUNK_E), :],
                                  attr_v.at[b], sem_s).wait()

    plsc.subcore_barrier()
    pltpu.sync_copy(acc.at[pl.ds(s * RPS, RPS), :],
                    partials.at[c, pl.ds(s * RPS, RPS), :])


CR = N_PAD // NW


@functools.partial(
    pl.kernel,
    out_type=jax.ShapeDtypeStruct((N_PAD, D_EDGE), jnp.float32),
    mesh=_mesh,
    compiler_params=pltpu.CompilerParams(use_tc_tiling_on_sc=False),
    scratch_types=[
        pltpu.VMEM((CR, D_EDGE), jnp.float32),
        pltpu.VMEM((CR, D_EDGE), jnp.float32),
        pltpu.SemaphoreType.DMA,
    ],
)
def _combine(partials, out, a_v, b_v, sem):
    c = lax.axis_index("c")
    s = lax.axis_index("s")
    w = s * NC + c
    r0 = w * CR

    pltpu.async_copy(partials.at[0, pl.ds(r0, CR), :], a_v, sem)
    pltpu.async_copy(partials.at[1, pl.ds(r0, CR), :], b_v, sem)
    pltpu.make_async_copy(partials.at[0, pl.ds(r0, CR), :], a_v, sem).wait()
    pltpu.make_async_copy(partials.at[1, pl.ds(r0, CR), :], b_v, sem).wait()

    def _add(i, carry):
        a_v[i, :] = a_v[i, :] + b_v[i, :]
        return carry

    lax.fori_loop(0, CR, _add, 0)
    pltpu.sync_copy(a_v, out.at[pl.ds(r0, CR), :])


def kernel(x, edge_index, edge_attr, u, batch):
    col = edge_index[1].astype(jnp.int32).reshape(G_TOTAL, B)
    partials = _scatter_partials(col, edge_attr)
    return _combine(partials)[:N_NODES]
